# baseline (speedup 1.0000x reference)
import functools

import jax
import jax.numpy as jnp
from jax import lax
from jax.experimental import pallas as pl
from jax.experimental.pallas import tpu as pltpu
from jax.experimental.pallas import tpu_sc as plsc

EMB = 32
CH = 128


def _sc_gather(u2, i2, Ugmf, Igmf, Umlp, Imlp, B, NC, NW, n_ch, b_per_w):
    mesh = plsc.VectorSubcoreMesh(core_axis_name="c", subcore_axis_name="s")
    out_t = tuple(jax.ShapeDtypeStruct((B, EMB), jnp.float32) for _ in range(4))

    @functools.partial(
        pl.kernel,
        mesh=mesh,
        out_type=out_t,
        compiler_params=pltpu.CompilerParams(use_tc_tiling_on_sc=False),
        scratch_types=[
            pltpu.VMEM((n_ch, CH), jnp.int32),
            pltpu.VMEM((n_ch, CH), jnp.int32),
            pltpu.VMEM((b_per_w, EMB), jnp.float32),
            pltpu.VMEM((b_per_w, EMB), jnp.float32),
            pltpu.VMEM((b_per_w, EMB), jnp.float32),
            pltpu.VMEM((b_per_w, EMB), jnp.float32),
            pltpu.SemaphoreType.DMA,
        ],
    )
    def gather_k(u_hbm, i_hbm, t0, t1, t2, t3, o0, o1, o2, o3,
                 ui_v, ii_v, r0, r1, r2, r3, sem):
        wid = lax.axis_index("s") * NC + lax.axis_index("c")
        base = wid * b_per_w
        pltpu.sync_copy(u_hbm.at[pl.ds(wid * n_ch, n_ch)], ui_v)
        pltpu.sync_copy(i_hbm.at[pl.ds(wid * n_ch, n_ch)], ii_v)
        copies = []
        for j in range(n_ch):
            dst = pl.ds(j * CH, CH)
            copies.append(pltpu.async_copy(t0.at[ui_v.at[j]], r0.at[dst], sem))
            copies.append(pltpu.async_copy(t1.at[ii_v.at[j]], r1.at[dst], sem))
            copies.append(pltpu.async_copy(t2.at[ui_v.at[j]], r2.at[dst], sem))
            copies.append(pltpu.async_copy(t3.at[ii_v.at[j]], r3.at[dst], sem))
        for c in copies:
            c.wait()
        pltpu.sync_copy(r0, o0.at[pl.ds(base, b_per_w)])
        pltpu.sync_copy(r1, o1.at[pl.ds(base, b_per_w)])
        pltpu.sync_copy(r2, o2.at[pl.ds(base, b_per_w)])
        pltpu.sync_copy(r3, o3.at[pl.ds(base, b_per_w)])

    return gather_k(u2, i2, Ugmf, Igmf, Umlp, Imlp)


def _mlp_body(ug_r, ig_r, um_r, im_r, w0_r, b0_r, w1_r, b1_r, w2_r, b2_r,
              wf_r, bf_r, out_r):
    gmf = ug_r[...] * ig_r[...]
    h = jnp.concatenate([um_r[...], im_r[...]], axis=1)
    h = jnp.maximum(
        jnp.dot(h, w0_r[...], preferred_element_type=jnp.float32) + b0_r[...], 0.0)
    h = jnp.maximum(
        jnp.dot(h, w1_r[...], preferred_element_type=jnp.float32) + b1_r[...], 0.0)
    h = jnp.maximum(
        jnp.dot(h, w2_r[...], preferred_element_type=jnp.float32) + b2_r[...], 0.0)
    cat = jnp.concatenate([gmf, h], axis=1)
    logit = jnp.dot(cat, wf_r[...], preferred_element_type=jnp.float32) + bf_r[...]
    out_r[...] = jax.nn.sigmoid(logit)


def _tc_mlp(ug, ig, um, im, W0, b0, W1, b1, W2, b2, Wf, bf, interpret=False):
    B = ug.shape[0]
    return pl.pallas_call(
        _mlp_body,
        out_shape=jax.ShapeDtypeStruct((B, 1), jnp.float32),
        interpret=interpret,
    )(ug, ig, um, im, W0, b0, W1, b1, W2, b2, Wf, bf)


def kernel(x, Ugmf, Igmf, Umlp, Imlp, W0, b0, W1, b1, W2, b2, Wf, bf):
    B = x.shape[0]
    info = plsc.get_sparse_core_info()
    NC, NS = info.num_cores, info.num_subcores
    NW = NC * NS
    b_per_w = B // NW
    n_ch = b_per_w // CH
    u2 = x[:, 0].reshape(NW * n_ch, CH).astype(jnp.int32)
    i2 = x[:, 1].reshape(NW * n_ch, CH).astype(jnp.int32)
    ug, ig, um, im = _sc_gather(u2, i2, Ugmf, Igmf, Umlp, Imlp,
                                B, NC, NW, n_ch, b_per_w)
    out = _tc_mlp(ug, ig, um, im, W0, b0, W1, b1, W2, b2, Wf, bf)
    return out[:, 0]


# speedup vs baseline: 1.3900x; 1.3900x over previous
import functools

import jax
import jax.numpy as jnp
from jax import lax
from jax.experimental import pallas as pl
from jax.experimental.pallas import tpu as pltpu
from jax.experimental.pallas import tpu_sc as plsc

EMB = 32
RCH = 16
BCH = 128


def _sc_gather(u2, i2, Ugmf, Igmf, Umlp, Imlp, B, NC, NW, n_ch, b_per_w):
    mesh = plsc.VectorSubcoreMesh(core_axis_name="c", subcore_axis_name="s")
    out_t = tuple(jax.ShapeDtypeStruct((B, EMB), jnp.float32) for _ in range(4))

    @functools.partial(
        pl.kernel,
        mesh=mesh,
        out_type=out_t,
        scratch_types=[
            pltpu.VMEM((b_per_w,), jnp.int32),
            pltpu.VMEM((b_per_w,), jnp.int32),
            pltpu.VMEM((BCH, EMB), jnp.float32),
            pltpu.VMEM((BCH, EMB), jnp.float32),
            pltpu.VMEM((BCH, EMB), jnp.float32),
            pltpu.VMEM((BCH, EMB), jnp.float32),
            pltpu.SemaphoreType.DMA,
        ],
    )
    def gather_k(u_hbm, i_hbm, t0, t1, t2, t3, o0, o1, o2, o3,
                 ui_v, ii_v, r0, r1, r2, r3, sem):
        wid = lax.axis_index("s") * NC + lax.axis_index("c")
        base = wid * b_per_w
        pltpu.sync_copy(u_hbm.at[pl.ds(base, b_per_w)], ui_v)
        pltpu.sync_copy(i_hbm.at[pl.ds(base, b_per_w)], ii_v)

        def big_chunk(c, _):
            cb = c * BCH

            def burst(d, _):
                vu = ui_v[pl.ds(cb + d * RCH, RCH)]
                vi = ii_v[pl.ds(cb + d * RCH, RCH)]
                copies = []
                for k in range(RCH):
                    iu = vu[k]
                    ii = vi[k]
                    dst = pl.ds(d * RCH + k, 1)
                    copies.append(
                        pltpu.async_copy(t0.at[pl.ds(iu, 1)], r0.at[dst], sem))
                    copies.append(
                        pltpu.async_copy(t1.at[pl.ds(ii, 1)], r1.at[dst], sem))
                    copies.append(
                        pltpu.async_copy(t2.at[pl.ds(iu, 1)], r2.at[dst], sem))
                    copies.append(
                        pltpu.async_copy(t3.at[pl.ds(ii, 1)], r3.at[dst], sem))
                for cp in copies:
                    cp.wait()
                return ()

            lax.fori_loop(0, BCH // RCH, burst, ())
            pltpu.sync_copy(r0, o0.at[pl.ds(base + cb, BCH)])
            pltpu.sync_copy(r1, o1.at[pl.ds(base + cb, BCH)])
            pltpu.sync_copy(r2, o2.at[pl.ds(base + cb, BCH)])
            pltpu.sync_copy(r3, o3.at[pl.ds(base + cb, BCH)])
            return ()

        lax.fori_loop(0, b_per_w // BCH, big_chunk, ())

    return gather_k(u2, i2, Ugmf, Igmf, Umlp, Imlp)


def _mlp_body(ug_r, ig_r, um_r, im_r, w0_r, b0_r, w1_r, b1_r, w2_r, b2_r,
              wf_r, bf_r, out_r):
    gmf = ug_r[...] * ig_r[...]
    h = jnp.concatenate([um_r[...], im_r[...]], axis=1)
    h = jnp.maximum(
        jnp.dot(h, w0_r[...], preferred_element_type=jnp.float32) + b0_r[...], 0.0)
    h = jnp.maximum(
        jnp.dot(h, w1_r[...], preferred_element_type=jnp.float32) + b1_r[...], 0.0)
    h = jnp.maximum(
        jnp.dot(h, w2_r[...], preferred_element_type=jnp.float32) + b2_r[...], 0.0)
    cat = jnp.concatenate([gmf, h], axis=1)
    logit = jnp.dot(cat, wf_r[...], preferred_element_type=jnp.float32) + bf_r[...]
    out_r[...] = jax.nn.sigmoid(logit)


def _tc_mlp(ug, ig, um, im, W0, b0, W1, b1, W2, b2, Wf, bf, interpret=False):
    B = ug.shape[0]
    return pl.pallas_call(
        _mlp_body,
        out_shape=jax.ShapeDtypeStruct((B, 1), jnp.float32),
        interpret=interpret,
    )(ug, ig, um, im, W0, b0, W1, b1, W2, b2, Wf, bf)


def kernel(x, Ugmf, Igmf, Umlp, Imlp, W0, b0, W1, b1, W2, b2, Wf, bf):
    B = x.shape[0]
    info = plsc.get_sparse_core_info()
    NC, NS = info.num_cores, info.num_subcores
    NW = NC * NS
    b_per_w = B // NW
    u2 = x[:, 0].astype(jnp.int32)
    i2 = x[:, 1].astype(jnp.int32)
    ug, ig, um, im = _sc_gather(u2, i2, Ugmf, Igmf, Umlp, Imlp,
                                B, NC, NW, 0, b_per_w)
    out = _tc_mlp(ug, ig, um, im, W0, b0, W1, b1, W2, b2, Wf, bf)
    return out[:, 0]
